# baseline (device time: 58424 ns/iter reference)
import jax
import jax.numpy as jnp
from jax import lax
from jax.experimental import pallas as pl
from jax.experimental.pallas import tpu as pltpu

N_DEV = 16
N_STAGES = 4

B, Sq, Hq, Hkv, Dh = 2, 128, 8, 2, 64
D = Hq * Dh
GROUP = Hq // Hkv
SCALE = 0.125


def kernel(x, Wq, Wo, K_ext, V_ext):
    skv_loc = K_ext.shape[1]

    def body(x_ref, wq_ref, wo_ref, k_ref, v_ref, out_ref,
             acc_ref, stats_ref, rbuf_ref, rstats_ref,
             send_o, recv_o, send_s, recv_s):
        my = lax.axis_index("i")

        for b in range(B):
            for h in range(Hq):
                q = lax.dot_general(
                    x_ref[b], wq_ref[:, h * Dh:(h + 1) * Dh],
                    (((1,), (0,)), ((), ())),
                )
                k = k_ref[b, :, h // GROUP, :]
                v = v_ref[b, :, h // GROUP, :]
                st = lax.dot_general(
                    k, q, (((1,), (1,)), ((), ()))
                ) * SCALE
                m = jnp.max(st, axis=0, keepdims=True)
                p = jnp.exp(st - m)
                l = jnp.sum(p, axis=0, keepdims=True)
                ot = lax.dot_general(
                    v, p, (((0,), (0,)), ((), ()))
                )
                acc_ref[b, h] = ot
                stats_ref[0, b * Hq + h] = m
                stats_ref[1, b * Hq + h] = l

        for s in range(N_STAGES):
            partner = my ^ (1 << s)
            out_rdma = pltpu.make_async_remote_copy(
                src_ref=acc_ref,
                dst_ref=rbuf_ref.at[s],
                send_sem=send_o.at[s],
                recv_sem=recv_o.at[s],
                device_id=(partner,),
                device_id_type=pl.DeviceIdType.MESH,
            )
            st_rdma = pltpu.make_async_remote_copy(
                src_ref=stats_ref,
                dst_ref=rstats_ref.at[s],
                send_sem=send_s.at[s],
                recv_sem=recv_s.at[s],
                device_id=(partner,),
                device_id_type=pl.DeviceIdType.MESH,
            )
            out_rdma.start()
            st_rdma.start()
            out_rdma.wait()
            st_rdma.wait()

            m_a = stats_ref[0]
            l_a = stats_ref[1]
            m_b = rstats_ref[s, 0]
            l_b = rstats_ref[s, 1]
            m_n = jnp.maximum(m_a, m_b)
            a_a = jnp.exp(m_a - m_n)
            a_b = jnp.exp(m_b - m_n)
            stats_ref[0] = m_n
            stats_ref[1] = l_a * a_a + l_b * a_b
            for b in range(B):
                for h in range(Hq):
                    bh = b * Hq + h
                    acc_ref[b, h] = (
                        acc_ref[b, h] * a_a[bh] + rbuf_ref[s, b, h] * a_b[bh]
                    )

        for b in range(B):
            acc_b = jnp.zeros((Sq, D), dtype=jnp.float32)
            for h in range(Hq):
                inv_l = 1.0 / stats_ref[1, b * Hq + h]
                o_n = acc_ref[b, h] * inv_l
                acc_b = acc_b + lax.dot_general(
                    o_n, wo_ref[h * Dh:(h + 1) * Dh, :],
                    (((0,), (0,)), ((), ())),
                )
            out_ref[b] = acc_b

    return pl.pallas_call(
        body,
        out_shape=jax.ShapeDtypeStruct((B, Sq, D), jnp.float32),
        in_specs=[pl.BlockSpec(memory_space=pltpu.VMEM)] * 5,
        out_specs=pl.BlockSpec(memory_space=pltpu.VMEM),
        scratch_shapes=[
            pltpu.VMEM((B, Hq, Dh, Sq), jnp.float32),
            pltpu.VMEM((2, B * Hq, 1, Sq), jnp.float32),
            pltpu.VMEM((N_STAGES, B, Hq, Dh, Sq), jnp.float32),
            pltpu.VMEM((N_STAGES, 2, B * Hq, 1, Sq), jnp.float32),
            pltpu.SemaphoreType.DMA((N_STAGES,)),
            pltpu.SemaphoreType.DMA((N_STAGES,)),
            pltpu.SemaphoreType.DMA((N_STAGES,)),
            pltpu.SemaphoreType.DMA((N_STAGES,)),
        ],
    )(x, Wq, Wo, K_ext, V_ext)


# device time: 44500 ns/iter; 1.3129x vs baseline; 1.3129x over previous
import jax
import jax.numpy as jnp
from jax import lax
from jax.experimental import pallas as pl
from jax.experimental.pallas import tpu as pltpu

N_DEV = 16
STAGE_MASKS = (1, 3, 4, 8)
N_STAGES = len(STAGE_MASKS)

B, Sq, Hq, Hkv, Dh = 2, 128, 8, 2, 64
D = Hq * Dh
GROUP = Hq // Hkv
SCALE = 0.125


def kernel(x, Wq, Wo, K_ext, V_ext):
    def body(x_ref, wq_ref, wo_ref, k_ref, v_ref, out_ref,
             acc_ref, stats_ref, sbuf_ref, rbuf_ref, rstats_ref,
             send_o, recv_o, send_s, recv_s):
        my = lax.axis_index("i")

        for b in range(B):
            for h in range(Hq):
                q = lax.dot_general(
                    x_ref[b], wq_ref[:, h * Dh:(h + 1) * Dh],
                    (((1,), (0,)), ((), ())),
                )
                k = k_ref[b, :, h // GROUP, :]
                v = v_ref[b, :, h // GROUP, :]
                st = lax.dot_general(
                    k, q, (((1,), (1,)), ((), ()))
                ) * SCALE
                m = jnp.max(st, axis=0, keepdims=True)
                p = jnp.exp(st - m)
                l = jnp.sum(p, axis=0, keepdims=True)
                ot = lax.dot_general(
                    v, p, (((0,), (0,)), ((), ()))
                )
                acc_ref[b, h] = ot
                stats_ref[0, b * Hq + h] = m
                stats_ref[1, b * Hq + h] = l

        for s, mask in enumerate(STAGE_MASKS):
            partner = my ^ mask
            for b in range(B):
                for h in range(Hq):
                    sbuf_ref[b, h] = acc_ref[b, h].astype(jnp.bfloat16)
            out_rdma = pltpu.make_async_remote_copy(
                src_ref=sbuf_ref,
                dst_ref=rbuf_ref.at[s],
                send_sem=send_o.at[s],
                recv_sem=recv_o.at[s],
                device_id=(partner,),
                device_id_type=pl.DeviceIdType.MESH,
            )
            st_rdma = pltpu.make_async_remote_copy(
                src_ref=stats_ref,
                dst_ref=rstats_ref.at[s],
                send_sem=send_s.at[s],
                recv_sem=recv_s.at[s],
                device_id=(partner,),
                device_id_type=pl.DeviceIdType.MESH,
            )
            out_rdma.start()
            st_rdma.start()

            st_rdma.wait()
            m_a = stats_ref[0]
            l_a = stats_ref[1]
            m_b = rstats_ref[s, 0]
            l_b = rstats_ref[s, 1]
            m_n = jnp.maximum(m_a, m_b)
            a_a = jnp.exp(m_a - m_n)
            a_b = jnp.exp(m_b - m_n)
            stats_ref[0] = m_n
            stats_ref[1] = l_a * a_a + l_b * a_b
            for b in range(B):
                for h in range(Hq):
                    acc_ref[b, h] = acc_ref[b, h] * a_a[b * Hq + h]

            out_rdma.wait()
            for b in range(B):
                for h in range(Hq):
                    acc_ref[b, h] = acc_ref[b, h] + (
                        rbuf_ref[s, b, h].astype(jnp.float32) * a_b[b * Hq + h]
                    )

        for b in range(B):
            acc_b = jnp.zeros((Sq, D), dtype=jnp.float32)
            for h in range(Hq):
                inv_l = 1.0 / stats_ref[1, b * Hq + h]
                o_n = acc_ref[b, h] * inv_l
                acc_b = acc_b + lax.dot_general(
                    o_n, wo_ref[h * Dh:(h + 1) * Dh, :],
                    (((0,), (0,)), ((), ())),
                )
            out_ref[b] = acc_b

    return pl.pallas_call(
        body,
        out_shape=jax.ShapeDtypeStruct((B, Sq, D), jnp.float32),
        in_specs=[pl.BlockSpec(memory_space=pltpu.VMEM)] * 5,
        out_specs=pl.BlockSpec(memory_space=pltpu.VMEM),
        scratch_shapes=[
            pltpu.VMEM((B, Hq, Dh, Sq), jnp.float32),
            pltpu.VMEM((2, B * Hq, 1, Sq), jnp.float32),
            pltpu.VMEM((B, Hq, Dh, Sq), jnp.bfloat16),
            pltpu.VMEM((N_STAGES, B, Hq, Dh, Sq), jnp.bfloat16),
            pltpu.VMEM((N_STAGES, 2, B * Hq, 1, Sq), jnp.float32),
            pltpu.SemaphoreType.DMA((N_STAGES,)),
            pltpu.SemaphoreType.DMA((N_STAGES,)),
            pltpu.SemaphoreType.DMA((N_STAGES,)),
            pltpu.SemaphoreType.DMA((N_STAGES,)),
        ],
    )(x, Wq, Wo, K_ext, V_ext)


# device time: 37684 ns/iter; 1.5504x vs baseline; 1.1809x over previous
import os

import jax
import jax.numpy as jnp
from jax import lax
from jax.experimental import pallas as pl
from jax.experimental.pallas import tpu as pltpu

_SKIP_COMM = os.environ.get("SKIP_COMM") == "1"

N_DEV = 16
STAGE_MASKS = (1, 3, 4, 8)
N_STAGES = len(STAGE_MASKS)

B, Sq, Hq, Hkv, Dh = 2, 128, 8, 2, 64
D = Hq * Dh
GROUP = Hq // Hkv
SCALE = 0.125


def kernel(x, Wq, Wo, K_ext, V_ext):
    def body(x_ref, wq_ref, wo_ref, k_ref, v_ref, out_ref,
             acc_ref, stats_ref, sbuf_ref, rbuf_ref, rstats_ref,
             send_o, recv_o, send_s, recv_s):
        my = lax.axis_index("i")

        if not _SKIP_COMM:
            barrier_sem = pltpu.get_barrier_semaphore()
            for mask in STAGE_MASKS:
                pl.semaphore_signal(
                    barrier_sem, inc=1,
                    device_id=(my ^ mask,),
                    device_id_type=pl.DeviceIdType.MESH,
                )

        for b in range(B):
            for h in range(Hq):
                q = lax.dot_general(
                    x_ref[b], wq_ref[:, h * Dh:(h + 1) * Dh],
                    (((1,), (0,)), ((), ())),
                )
                k = k_ref[b, :, h // GROUP, :]
                v = v_ref[b, :, h // GROUP, :]
                st = lax.dot_general(
                    k, q, (((1,), (1,)), ((), ()))
                ) * SCALE
                m = jnp.max(st, axis=0, keepdims=True)
                p = jnp.exp(st - m)
                l = jnp.sum(p, axis=0, keepdims=True)
                ot = lax.dot_general(
                    v, p, (((0,), (0,)), ((), ()))
                )
                acc_ref[b, h] = ot
                stats_ref[0, b, h] = m
                stats_ref[1, b, h] = l

        if not _SKIP_COMM:
            pl.semaphore_wait(barrier_sem, N_STAGES)

        for s, mask in enumerate(() if _SKIP_COMM else STAGE_MASKS):
            partner = my ^ mask
            sbuf_ref[...] = acc_ref[...].astype(jnp.bfloat16)
            out_rdma = pltpu.make_async_remote_copy(
                src_ref=sbuf_ref,
                dst_ref=rbuf_ref.at[s],
                send_sem=send_o.at[s],
                recv_sem=recv_o.at[s],
                device_id=(partner,),
                device_id_type=pl.DeviceIdType.MESH,
            )
            st_rdma = pltpu.make_async_remote_copy(
                src_ref=stats_ref,
                dst_ref=rstats_ref.at[s],
                send_sem=send_s.at[s],
                recv_sem=recv_s.at[s],
                device_id=(partner,),
                device_id_type=pl.DeviceIdType.MESH,
            )
            out_rdma.start()
            st_rdma.start()

            st_rdma.wait()
            m_a = stats_ref[0]
            l_a = stats_ref[1]
            m_b = rstats_ref[s, 0]
            l_b = rstats_ref[s, 1]
            m_n = jnp.maximum(m_a, m_b)
            a_a = jnp.exp(m_a - m_n)
            a_b = jnp.exp(m_b - m_n)
            stats_ref[0] = m_n
            stats_ref[1] = l_a * a_a + l_b * a_b
            acc_ref[...] = acc_ref[...] * a_a

            out_rdma.wait()
            acc_ref[...] = acc_ref[...] + rbuf_ref[s].astype(jnp.float32) * a_b

        for b in range(B):
            acc_b = jnp.zeros((Sq, D), dtype=jnp.float32)
            for h in range(Hq):
                inv_l = 1.0 / stats_ref[1, b, h]
                o_n = acc_ref[b, h] * inv_l
                acc_b = acc_b + lax.dot_general(
                    o_n, wo_ref[h * Dh:(h + 1) * Dh, :],
                    (((0,), (0,)), ((), ())),
                )
            out_ref[b] = acc_b

    return pl.pallas_call(
        body,
        out_shape=jax.ShapeDtypeStruct((B, Sq, D), jnp.float32),
        in_specs=[pl.BlockSpec(memory_space=pltpu.VMEM)] * 5,
        out_specs=pl.BlockSpec(memory_space=pltpu.VMEM),
        scratch_shapes=[
            pltpu.VMEM((B, Hq, Dh, Sq), jnp.float32),
            pltpu.VMEM((2, B, Hq, 1, Sq), jnp.float32),
            pltpu.VMEM((B, Hq, Dh, Sq), jnp.bfloat16),
            pltpu.VMEM((N_STAGES, B, Hq, Dh, Sq), jnp.bfloat16),
            pltpu.VMEM((N_STAGES, 2, B, Hq, 1, Sq), jnp.float32),
            pltpu.SemaphoreType.DMA((N_STAGES,)),
            pltpu.SemaphoreType.DMA((N_STAGES,)),
            pltpu.SemaphoreType.DMA((N_STAGES,)),
            pltpu.SemaphoreType.DMA((N_STAGES,)),
        ],
        compiler_params=pltpu.CompilerParams(collective_id=0),
    )(x, Wq, Wo, K_ext, V_ext)


# device time: 30337 ns/iter; 1.9258x vs baseline; 1.2422x over previous
import os

import jax
import jax.numpy as jnp
from jax import lax
from jax.experimental import pallas as pl
from jax.experimental.pallas import tpu as pltpu

_SKIP_COMM = os.environ.get("SKIP_COMM") == "1"

N_DEV = 16
STAGE_MASKS = (1, 3, 4, 8)
N_STAGES = len(STAGE_MASKS)

B, Sq, Hq, Hkv, Dh = 2, 128, 8, 2, 64
D = Hq * Dh
GROUP = Hq // Hkv
SCALE = 0.125


def kernel(x, Wq, Wo, K_ext, V_ext):
    def body(x_ref, wq_ref, wo_ref, k_ref, v_ref, out_ref,
             acc_ref, stats_ref, sbuf_ref, rbuf_ref, rstats_ref,
             send_o, recv_o, send_s, recv_s):
        my = lax.axis_index("i")

        if not _SKIP_COMM:
            barrier_sem = pltpu.get_barrier_semaphore()
            for mask in STAGE_MASKS:
                pl.semaphore_signal(
                    barrier_sem, inc=1,
                    device_id=(my ^ mask,),
                    device_id_type=pl.DeviceIdType.MESH,
                )

        def partial(b):
            for h in range(Hq):
                q = lax.dot_general(
                    x_ref[b], wq_ref[:, h * Dh:(h + 1) * Dh],
                    (((1,), (0,)), ((), ())),
                )
                k = k_ref[b, :, h // GROUP, :]
                v = v_ref[b, :, h // GROUP, :]
                st = lax.dot_general(
                    k, q, (((1,), (1,)), ((), ()))
                ) * SCALE
                m = jnp.max(st, axis=0, keepdims=True)
                p = jnp.exp(st - m)
                l = jnp.sum(p, axis=0, keepdims=True)
                ot = lax.dot_general(
                    v, p, (((0,), (0,)), ((), ()))
                )
                acc_ref[b, h] = ot
                stats_ref[b, 0, h] = m
                stats_ref[b, 1, h] = l

        def send(s, t):
            partner = my ^ STAGE_MASKS[s]
            st_rdma = pltpu.make_async_remote_copy(
                src_ref=stats_ref.at[t],
                dst_ref=rstats_ref.at[s * B + t],
                send_sem=send_s.at[s, t],
                recv_sem=recv_s.at[s, t],
                device_id=(partner,),
                device_id_type=pl.DeviceIdType.MESH,
            )
            sbuf_ref[t] = acc_ref[t].astype(jnp.bfloat16)
            out_rdma = pltpu.make_async_remote_copy(
                src_ref=sbuf_ref.at[t],
                dst_ref=rbuf_ref.at[s * B + t],
                send_sem=send_o.at[s, t],
                recv_sem=recv_o.at[s, t],
                device_id=(partner,),
                device_id_type=pl.DeviceIdType.MESH,
            )
            st_rdma.start()
            out_rdma.start()
            return st_rdma, out_rdma

        def combine(s, t, rdmas):
            st_rdma, out_rdma = rdmas
            st_rdma.wait()
            m_a = stats_ref[t, 0]
            l_a = stats_ref[t, 1]
            m_b = rstats_ref[s * B + t, 0]
            l_b = rstats_ref[s * B + t, 1]
            m_n = jnp.maximum(m_a, m_b)
            a_a = jnp.exp(m_a - m_n)
            a_b = jnp.exp(m_b - m_n)
            stats_ref[t, 0] = m_n
            stats_ref[t, 1] = l_a * a_a + l_b * a_b
            acc_ref[t] = acc_ref[t] * a_a
            out_rdma.wait()
            acc_ref[t] = acc_ref[t] + (
                rbuf_ref[s * B + t].astype(jnp.float32) * a_b
            )

        def project(b):
            acc_b = jnp.zeros((Sq, D), dtype=jnp.float32)
            for h in range(Hq):
                inv_l = 1.0 / stats_ref[b, 1, h]
                o_n = acc_ref[b, h] * inv_l
                acc_b = acc_b + lax.dot_general(
                    o_n, wo_ref[h * Dh:(h + 1) * Dh, :],
                    (((0,), (0,)), ((), ())),
                )
            out_ref[b] = acc_b

        if _SKIP_COMM:
            partial(0)
            partial(1)
            project(0)
            project(1)
            return

        partial(0)
        pl.semaphore_wait(barrier_sem, N_STAGES)
        r = [[None, None] for _ in range(N_STAGES)]
        r[0][0] = send(0, 0)
        partial(1)
        r[0][1] = send(0, 1)
        for s in range(N_STAGES - 1):
            combine(s, 0, r[s][0])
            r[s + 1][0] = send(s + 1, 0)
            combine(s, 1, r[s][1])
            r[s + 1][1] = send(s + 1, 1)
        combine(N_STAGES - 1, 0, r[N_STAGES - 1][0])
        project(0)
        combine(N_STAGES - 1, 1, r[N_STAGES - 1][1])
        project(1)

    return pl.pallas_call(
        body,
        out_shape=jax.ShapeDtypeStruct((B, Sq, D), jnp.float32),
        in_specs=[pl.BlockSpec(memory_space=pltpu.VMEM)] * 5,
        out_specs=pl.BlockSpec(memory_space=pltpu.VMEM),
        scratch_shapes=[
            pltpu.VMEM((B, Hq, Dh, Sq), jnp.float32),
            pltpu.VMEM((B, 2, Hq, 1, Sq), jnp.float32),
            pltpu.VMEM((B, Hq, Dh, Sq), jnp.bfloat16),
            pltpu.VMEM((N_STAGES * B, Hq, Dh, Sq), jnp.bfloat16),
            pltpu.VMEM((N_STAGES * B, 2, Hq, 1, Sq), jnp.float32),
            pltpu.SemaphoreType.DMA((N_STAGES, B)),
            pltpu.SemaphoreType.DMA((N_STAGES, B)),
            pltpu.SemaphoreType.DMA((N_STAGES, B)),
            pltpu.SemaphoreType.DMA((N_STAGES, B)),
        ],
        compiler_params=pltpu.CompilerParams(collective_id=0),
    )(x, Wq, Wo, K_ext, V_ext)


# device time: 27649 ns/iter; 2.1131x vs baseline; 1.0972x over previous
import os

import jax
import jax.numpy as jnp
from jax import lax
from jax.experimental import pallas as pl
from jax.experimental.pallas import tpu as pltpu

_SKIP_COMM = os.environ.get("SKIP_COMM") == "1"

N_DEV = 16
STAGE_MASKS = (1, 3, 4, 8)
N_STAGES = len(STAGE_MASKS)

B, Sq, Hq, Hkv, Dh = 2, 128, 8, 2, 64
D = Hq * Dh
GROUP = Hq // Hkv
GSQ = GROUP * Sq
SCALE = 0.125


def kernel(x, Wq, Wo, K_ext, V_ext):
    def body(x_ref, wq_ref, wo_ref, k_ref, v_ref, out_ref,
             acc_ref, stats_ref, sbuf_ref, rbuf_ref, rstats_ref,
             kbuf_ref, vbuf_ref, qg_ref, obuf_ref,
             send_o, recv_o, send_s, recv_s):
        my = lax.axis_index("i")

        if not _SKIP_COMM:
            barrier_sem = pltpu.get_barrier_semaphore()
            for mask in STAGE_MASKS:
                pl.semaphore_signal(
                    barrier_sem, inc=1,
                    device_id=(my ^ mask,),
                    device_id_type=pl.DeviceIdType.MESH,
                )

        def partial(b):
            for g in range(Hkv):
                kbuf_ref[b, g] = k_ref[b, :, g, :]
                vbuf_ref[b, g] = v_ref[b, :, g, :]
            q_all = lax.dot_general(
                x_ref[b], wq_ref[...], (((1,), (0,)), ((), ())),
            )
            for h in range(Hq):
                g, hh = divmod(h, GROUP)
                qg_ref[b, g, hh * Sq:(hh + 1) * Sq, :] = (
                    q_all[:, h * Dh:(h + 1) * Dh]
                )
            st = lax.dot_general(
                kbuf_ref[b], qg_ref[b], (((2,), (2,)), ((0,), (0,))),
            ) * SCALE
            m = jnp.max(st, axis=1, keepdims=True)
            p = jnp.exp(st - m)
            l = jnp.sum(p, axis=1, keepdims=True)
            ot = lax.dot_general(
                vbuf_ref[b], p, (((1,), (1,)), ((0,), (0,))),
            )
            acc_ref[b] = ot
            stats_ref[b, 0] = m
            stats_ref[b, 1] = l

        def send(s, t):
            partner = my ^ STAGE_MASKS[s]
            st_rdma = pltpu.make_async_remote_copy(
                src_ref=stats_ref.at[t],
                dst_ref=rstats_ref.at[s * B + t],
                send_sem=send_s.at[s, t],
                recv_sem=recv_s.at[s, t],
                device_id=(partner,),
                device_id_type=pl.DeviceIdType.MESH,
            )
            sbuf_ref[t] = acc_ref[t].astype(jnp.bfloat16)
            out_rdma = pltpu.make_async_remote_copy(
                src_ref=sbuf_ref.at[t],
                dst_ref=rbuf_ref.at[s * B + t],
                send_sem=send_o.at[s, t],
                recv_sem=recv_o.at[s, t],
                device_id=(partner,),
                device_id_type=pl.DeviceIdType.MESH,
            )
            st_rdma.start()
            out_rdma.start()
            return st_rdma, out_rdma

        def combine(s, t, rdmas):
            st_rdma, out_rdma = rdmas
            st_rdma.wait()
            m_a = stats_ref[t, 0]
            l_a = stats_ref[t, 1]
            m_b = rstats_ref[s * B + t, 0]
            l_b = rstats_ref[s * B + t, 1]
            m_n = jnp.maximum(m_a, m_b)
            a_a = jnp.exp(m_a - m_n)
            a_b = jnp.exp(m_b - m_n)
            stats_ref[t, 0] = m_n
            stats_ref[t, 1] = l_a * a_a + l_b * a_b
            acc_ref[t] = acc_ref[t] * a_a
            out_rdma.wait()
            acc_ref[t] = acc_ref[t] + (
                rbuf_ref[s * B + t].astype(jnp.float32) * a_b
            )

        def project(b):
            for h in range(Hq):
                g, hh = divmod(h, GROUP)
                inv_l = 1.0 / stats_ref[b, 1, g, :, hh * Sq:(hh + 1) * Sq]
                obuf_ref[h * Dh:(h + 1) * Dh, :] = (
                    acc_ref[b, g, :, hh * Sq:(hh + 1) * Sq] * inv_l
                )
            out_ref[b] = lax.dot_general(
                obuf_ref[...], wo_ref[...], (((0,), (0,)), ((), ())),
            )

        if _SKIP_COMM:
            partial(0)
            partial(1)
            project(0)
            project(1)
            return

        partial(0)
        pl.semaphore_wait(barrier_sem, N_STAGES)
        r = [[None, None] for _ in range(N_STAGES)]
        r[0][0] = send(0, 0)
        partial(1)
        r[0][1] = send(0, 1)
        for s in range(N_STAGES - 1):
            combine(s, 0, r[s][0])
            r[s + 1][0] = send(s + 1, 0)
            combine(s, 1, r[s][1])
            r[s + 1][1] = send(s + 1, 1)
        combine(N_STAGES - 1, 0, r[N_STAGES - 1][0])
        project(0)
        combine(N_STAGES - 1, 1, r[N_STAGES - 1][1])
        project(1)

    return pl.pallas_call(
        body,
        out_shape=jax.ShapeDtypeStruct((B, Sq, D), jnp.float32),
        in_specs=[pl.BlockSpec(memory_space=pltpu.VMEM)] * 5,
        out_specs=pl.BlockSpec(memory_space=pltpu.VMEM),
        scratch_shapes=[
            pltpu.VMEM((B, Hkv, Dh, GSQ), jnp.float32),
            pltpu.VMEM((B, 2, Hkv, 1, GSQ), jnp.float32),
            pltpu.VMEM((B, Hkv, Dh, GSQ), jnp.bfloat16),
            pltpu.VMEM((N_STAGES * B, Hkv, Dh, GSQ), jnp.bfloat16),
            pltpu.VMEM((N_STAGES * B, 2, Hkv, 1, GSQ), jnp.float32),
            pltpu.VMEM((B, Hkv, Sq, Dh), jnp.float32),
            pltpu.VMEM((B, Hkv, Sq, Dh), jnp.float32),
            pltpu.VMEM((B, Hkv, GSQ, Dh), jnp.float32),
            pltpu.VMEM((D, Sq), jnp.float32),
            pltpu.SemaphoreType.DMA((N_STAGES, B)),
            pltpu.SemaphoreType.DMA((N_STAGES, B)),
            pltpu.SemaphoreType.DMA((N_STAGES, B)),
            pltpu.SemaphoreType.DMA((N_STAGES, B)),
        ],
        compiler_params=pltpu.CompilerParams(collective_id=0),
    )(x, Wq, Wo, K_ext, V_ext)
